# baseline (speedup 1.0000x reference)
import functools

import jax
import jax.numpy as jnp
from jax import lax
from jax.experimental import pallas as pl
from jax.experimental.pallas import tpu as pltpu
from jax.experimental.pallas import tpu_sc as plsc

_N = 10000
_E = 320000
_DIN = 128
_DHID = 16
_NCLS = 10

_NPAD = 10240
_RB = 1024
_GRID = _NPAD // _RB

_NCORES = 2
_NSUB = 16
_NW = _NCORES * _NSUB
_CH = 128
_NCH = 80
_EPAD = _NW * _NCH * _CH
_RPT = _NPAD // _NSUB
_DEGW = 8


def _mesh():
    return plsc.VectorSubcoreMesh(
        core_axis_name="c", subcore_axis_name="s",
        num_cores=_NCORES, num_subcores=_NSUB)


def _sc_degree(dst2d, ones, zeros8):
    @functools.partial(
        pl.kernel, mesh=_mesh(),
        compiler_params=pltpu.CompilerParams(use_tc_tiling_on_sc=False),
        out_type=jax.ShapeDtypeStruct((_NCORES, _NPAD, _DEGW), jnp.float32),
        scratch_types=[
            pltpu.VMEM((_NCH, _CH), jnp.int32),
            pltpu.VMEM((_CH, _DEGW), jnp.float32),
            pltpu.VMEM_SHARED((_NPAD, _DEGW), jnp.float32),
        ])
    def deg_kernel(dst_hbm, ones_hbm, z_hbm, out_hbm, didx, ones_v, acc):
        cid = lax.axis_index("c")
        sid = lax.axis_index("s")
        wid = sid * _NCORES + cid
        pltpu.sync_copy(z_hbm, acc.at[pl.ds(sid * _RPT, _RPT)])
        pltpu.sync_copy(ones_hbm, ones_v)
        pltpu.sync_copy(dst_hbm.at[pl.ds(wid * _NCH, _NCH)], didx)
        plsc.subcore_barrier()

        def chunk(j, c):
            pltpu.sync_copy(ones_v, acc.at[didx.at[j]], add=True)
            return c

        lax.fori_loop(0, _NCH, chunk, None)
        plsc.subcore_barrier()
        pltpu.sync_copy(acc.at[pl.ds(sid * _RPT, _RPT)],
                        out_hbm.at[cid, pl.ds(sid * _RPT, _RPT)])

    return deg_kernel(dst2d, ones, zeros8)


def _sc_scatter(hh, src2d, dst2d, zeros16):
    @functools.partial(
        pl.kernel, mesh=_mesh(),
        compiler_params=pltpu.CompilerParams(use_tc_tiling_on_sc=False),
        out_type=jax.ShapeDtypeStruct((_NCORES, _NPAD, _DHID), jnp.float32),
        scratch_types=[
            pltpu.VMEM((_NCH, _CH), jnp.int32),
            pltpu.VMEM((_NCH, _CH), jnp.int32),
            pltpu.VMEM((2, _CH, _DHID), jnp.float32),
            pltpu.VMEM_SHARED((_NPAD, _DHID), jnp.float32),
            pltpu.SemaphoreType.DMA,
            pltpu.SemaphoreType.DMA,
        ])
    def scat_kernel(hh_hbm, src_hbm, dst_hbm, z_hbm, out_hbm,
                    sidx, didx, rows, acc, sem_a, sem_b):
        cid = lax.axis_index("c")
        sid = lax.axis_index("s")
        wid = sid * _NCORES + cid
        pltpu.sync_copy(z_hbm, acc.at[pl.ds(sid * _RPT, _RPT)])
        pltpu.sync_copy(src_hbm.at[pl.ds(wid * _NCH, _NCH)], sidx)
        pltpu.sync_copy(dst_hbm.at[pl.ds(wid * _NCH, _NCH)], didx)
        plsc.subcore_barrier()

        pltpu.async_copy(hh_hbm.at[sidx.at[0]], rows.at[0], sem_a)

        def pair(j, c):
            e = 2 * j
            pltpu.async_copy(hh_hbm.at[sidx.at[e + 1]], rows.at[1], sem_b)
            pltpu.make_async_copy(hh_hbm.at[sidx.at[e]], rows.at[0],
                                  sem_a).wait()
            pltpu.sync_copy(rows.at[0], acc.at[didx.at[e]], add=True)

            @pl.when(e + 2 < _NCH)
            def _():
                pltpu.async_copy(hh_hbm.at[sidx.at[e + 2]], rows.at[0],
                                 sem_a)

            pltpu.make_async_copy(hh_hbm.at[sidx.at[e + 1]], rows.at[1],
                                  sem_b).wait()
            pltpu.sync_copy(rows.at[1], acc.at[didx.at[e + 1]], add=True)
            return c

        lax.fori_loop(0, _NCH // 2, pair, None)
        plsc.subcore_barrier()
        pltpu.sync_copy(acc.at[pl.ds(sid * _RPT, _RPT)],
                        out_hbm.at[cid, pl.ds(sid * _RPT, _RPT)])

    return scat_kernel(hh, src2d, dst2d, zeros16)


def _dinv_block(deg_ref):
    dsum = deg_ref[0] + deg_ref[1] + 1.0
    return lax.rsqrt(jnp.maximum(dsum, 1.0))[:, :1]


def _tc1_body(x_ref, w1_ref, deg_ref, o_ref):
    dinv = _dinv_block(deg_ref)
    h = jnp.dot(x_ref[...], w1_ref[...], preferred_element_type=jnp.float32)
    o_ref[...] = h * dinv


def _tc1(x, W1, deg_parts):
    return pl.pallas_call(
        _tc1_body,
        grid=(_GRID,),
        in_specs=[
            pl.BlockSpec((_RB, _DIN), lambda i: (i, 0)),
            pl.BlockSpec((_DIN, _DHID), lambda i: (0, 0)),
            pl.BlockSpec((_NCORES, _RB, _DEGW), lambda i: (0, i, 0)),
        ],
        out_specs=pl.BlockSpec((_RB, _DHID), lambda i: (i, 0)),
        out_shape=jax.ShapeDtypeStruct((_N, _DHID), jnp.float32),
    )(x, W1, deg_parts)


def _tc2_body(acc_ref, hh_ref, deg_ref, w2_ref, b1_ref, o_ref):
    dinv = _dinv_block(deg_ref)
    s = acc_ref[0] + acc_ref[1] + hh_ref[...]
    h1 = jnp.maximum(s * dinv + b1_ref[...], 0.0)
    h2 = jnp.dot(h1, w2_ref[...], preferred_element_type=jnp.float32)
    o_ref[...] = h2 * dinv


def _tc2(acc1, hh, deg_parts, W2p, b1r):
    return pl.pallas_call(
        _tc2_body,
        grid=(_GRID,),
        in_specs=[
            pl.BlockSpec((_NCORES, _RB, _DHID), lambda i: (0, i, 0)),
            pl.BlockSpec((_RB, _DHID), lambda i: (i, 0)),
            pl.BlockSpec((_NCORES, _RB, _DEGW), lambda i: (0, i, 0)),
            pl.BlockSpec((_DHID, _DHID), lambda i: (0, 0)),
            pl.BlockSpec((1, _DHID), lambda i: (0, 0)),
        ],
        out_specs=pl.BlockSpec((_RB, _DHID), lambda i: (i, 0)),
        out_shape=jax.ShapeDtypeStruct((_N, _DHID), jnp.float32),
    )(acc1, hh, deg_parts, W2p, b1r)


def _tc3_body(acc_ref, hh_ref, deg_ref, b2_ref, o_ref):
    dinv = _dinv_block(deg_ref)
    s = acc_ref[0] + acc_ref[1] + hh_ref[...]
    z = s * dinv + b2_ref[...]
    col = lax.broadcasted_iota(jnp.int32, (_RB, _DHID), 1)
    mask = col < _NCLS
    neg = jnp.full_like(z, -3.0e38)
    m = jnp.max(jnp.where(mask, z, neg), axis=1, keepdims=True)
    e = jnp.where(mask, jnp.exp(z - m), 0.0)
    lse = jnp.log(jnp.sum(e, axis=1, keepdims=True))
    o_ref[...] = (z - m - lse)[:, :_NCLS]


def _tc3(acc2, hh2, deg_parts, b2r):
    return pl.pallas_call(
        _tc3_body,
        grid=(_GRID,),
        in_specs=[
            pl.BlockSpec((_NCORES, _RB, _DHID), lambda i: (0, i, 0)),
            pl.BlockSpec((_RB, _DHID), lambda i: (i, 0)),
            pl.BlockSpec((_NCORES, _RB, _DEGW), lambda i: (0, i, 0)),
            pl.BlockSpec((1, _DHID), lambda i: (0, 0)),
        ],
        out_specs=pl.BlockSpec((_RB, _NCLS), lambda i: (i, 0)),
        out_shape=jax.ShapeDtypeStruct((_N, _NCLS), jnp.float32),
    )(acc2, hh2, deg_parts, b2r)


def kernel(x, edge_index, W1, b1, W2, b2):
    npad_e = _EPAD - _E
    pad_src = jnp.zeros((npad_e,), jnp.int32)
    pad_dst = _N + (jnp.arange(npad_e, dtype=jnp.int32) % (_NPAD - _N))
    src2d = jnp.concatenate(
        [edge_index[0].astype(jnp.int32), pad_src]).reshape(_NW * _NCH, _CH)
    dst2d = jnp.concatenate(
        [edge_index[1].astype(jnp.int32), pad_dst]).reshape(_NW * _NCH, _CH)
    W2p = jnp.pad(W2, ((0, 0), (0, _DHID - _NCLS)))
    b1r = b1.reshape(1, _DHID)
    b2r = jnp.pad(b2, (0, _DHID - _NCLS)).reshape(1, _DHID)
    ones = jnp.ones((_CH, _DEGW), jnp.float32)
    zeros8 = jnp.zeros((_RPT, _DEGW), jnp.float32)
    zeros16 = jnp.zeros((_RPT, _DHID), jnp.float32)

    deg_parts = _sc_degree(dst2d, ones, zeros8)
    hh = _tc1(x, W1, deg_parts)
    acc1 = _sc_scatter(hh, src2d, dst2d, zeros16)
    hh2 = _tc2(acc1, hh, deg_parts, W2p, b1r)
    acc2 = _sc_scatter(hh2, src2d, dst2d, zeros16)
    return _tc3(acc2, hh2, deg_parts, b2r)


# speedup vs baseline: 1.1284x; 1.1284x over previous
import functools

import jax
import jax.numpy as jnp
from jax import lax
from jax.experimental import pallas as pl
from jax.experimental.pallas import tpu as pltpu
from jax.experimental.pallas import tpu_sc as plsc

_N = 10000
_E = 320000
_DIN = 128
_DHID = 16
_NCLS = 10

_NPAD = 10240
_RB = 1024
_GRID = _NPAD // _RB

_NCORES = 2
_NSUB = 16
_NW = _NCORES * _NSUB
_CH = 72
_NCH = 140
_EPAD = _NW * _NCH * _CH
_RPT = _NPAD // _NSUB
_DEGW = 8


def _mesh():
    return plsc.VectorSubcoreMesh(
        core_axis_name="c", subcore_axis_name="s",
        num_cores=_NCORES, num_subcores=_NSUB)


def _sc_degree(dst2d, ones, zeros8):
    @functools.partial(
        pl.kernel, mesh=_mesh(),
        compiler_params=pltpu.CompilerParams(use_tc_tiling_on_sc=False),
        out_type=jax.ShapeDtypeStruct((_NCORES, _NPAD, _DEGW), jnp.float32),
        scratch_types=[
            pltpu.VMEM((_NCH, _CH), jnp.int32),
            pltpu.VMEM((_CH, _DEGW), jnp.float32),
            pltpu.VMEM_SHARED((_NPAD, _DEGW), jnp.float32),
        ])
    def deg_kernel(dst_hbm, ones_hbm, z_hbm, out_hbm, didx, ones_v, acc):
        cid = lax.axis_index("c")
        sid = lax.axis_index("s")
        wid = sid * _NCORES + cid
        pltpu.sync_copy(z_hbm, acc.at[pl.ds(sid * _RPT, _RPT)])
        pltpu.sync_copy(ones_hbm, ones_v)
        pltpu.sync_copy(dst_hbm.at[pl.ds(wid * _NCH, _NCH)], didx)
        plsc.subcore_barrier()

        def chunk(j, c):
            pltpu.sync_copy(ones_v, acc.at[didx.at[j]], add=True)
            return c

        lax.fori_loop(0, _NCH, chunk, None)
        plsc.subcore_barrier()
        pltpu.sync_copy(acc.at[pl.ds(sid * _RPT, _RPT)],
                        out_hbm.at[cid, pl.ds(sid * _RPT, _RPT)])

    return deg_kernel(dst2d, ones, zeros8)


def _sc_scatter(hh, src2d, dst2d, zeros16):
    @functools.partial(
        pl.kernel, mesh=_mesh(),
        compiler_params=pltpu.CompilerParams(use_tc_tiling_on_sc=False),
        out_type=jax.ShapeDtypeStruct((_NCORES, _NPAD, _DHID), jnp.float32),
        scratch_types=[
            pltpu.VMEM((_NCH, _CH), jnp.int32),
            pltpu.VMEM((_NCH, _CH), jnp.int32),
            pltpu.VMEM((4, _CH, _DHID), jnp.float32),
            pltpu.VMEM_SHARED((_NPAD, _DHID), jnp.float32),
            [pltpu.SemaphoreType.DMA] * 4,
            [pltpu.SemaphoreType.DMA] * 4,
        ])
    def scat_kernel(hh_hbm, src_hbm, dst_hbm, z_hbm, out_hbm,
                    sidx, didx, rows, acc, gsem, ssem):
        cid = lax.axis_index("c")
        sid = lax.axis_index("s")
        wid = sid * _NCORES + cid
        pltpu.sync_copy(z_hbm, acc.at[pl.ds(sid * _RPT, _RPT)])
        pltpu.sync_copy(src_hbm.at[pl.ds(wid * _NCH, _NCH)], sidx)
        pltpu.sync_copy(dst_hbm.at[pl.ds(wid * _NCH, _NCH)], didx)
        plsc.subcore_barrier()

        def gat(c, k):
            return pltpu.make_async_copy(hh_hbm.at[sidx.at[c]], rows.at[k],
                                         gsem[k])

        def sca(c, k):
            return pltpu.make_async_copy(rows.at[k], acc.at[didx.at[c]],
                                         ssem[k])

        for k in range(4):
            pltpu.async_copy(hh_hbm.at[sidx.at[k]], rows.at[k], gsem[k])

        def quad(j, carry):
            c0 = 4 * j
            for k in range(4):
                gat(c0 + k, k).wait()
                pltpu.async_copy(rows.at[k], acc.at[didx.at[c0 + k]],
                                 ssem[k], add=True)
            for k in range(4):
                sca(c0 + k, k).wait()
                pltpu.async_copy(hh_hbm.at[sidx.at[c0 + 4 + k]], rows.at[k],
                                 gsem[k])
            return carry

        lax.fori_loop(0, _NCH // 4 - 1, quad, None)
        c0 = _NCH - 4
        for k in range(4):
            gat(c0 + k, k).wait()
            pltpu.async_copy(rows.at[k], acc.at[didx.at[c0 + k]],
                             ssem[k], add=True)
        for k in range(4):
            sca(c0 + k, k).wait()
        plsc.subcore_barrier()
        pltpu.sync_copy(acc.at[pl.ds(sid * _RPT, _RPT)],
                        out_hbm.at[cid, pl.ds(sid * _RPT, _RPT)])

    return scat_kernel(hh, src2d, dst2d, zeros16)


def _dinv_block(deg_ref):
    dsum = deg_ref[0] + deg_ref[1] + 1.0
    return lax.rsqrt(jnp.maximum(dsum, 1.0))[:, :1]


def _tc1_body(x_ref, w1_ref, deg_ref, o_ref):
    dinv = _dinv_block(deg_ref)
    h = jnp.dot(x_ref[...], w1_ref[...], preferred_element_type=jnp.float32)
    o_ref[...] = h * dinv


def _tc1(x, W1, deg_parts):
    return pl.pallas_call(
        _tc1_body,
        grid=(_GRID,),
        in_specs=[
            pl.BlockSpec((_RB, _DIN), lambda i: (i, 0)),
            pl.BlockSpec((_DIN, _DHID), lambda i: (0, 0)),
            pl.BlockSpec((_NCORES, _RB, _DEGW), lambda i: (0, i, 0)),
        ],
        out_specs=pl.BlockSpec((_RB, _DHID), lambda i: (i, 0)),
        out_shape=jax.ShapeDtypeStruct((_N, _DHID), jnp.float32),
    )(x, W1, deg_parts)


def _tc2_body(acc_ref, hh_ref, deg_ref, w2_ref, b1_ref, o_ref):
    dinv = _dinv_block(deg_ref)
    s = acc_ref[0] + acc_ref[1] + hh_ref[...]
    h1 = jnp.maximum(s * dinv + b1_ref[...], 0.0)
    h2 = jnp.dot(h1, w2_ref[...], preferred_element_type=jnp.float32)
    o_ref[...] = h2 * dinv


def _tc2(acc1, hh, deg_parts, W2p, b1r):
    return pl.pallas_call(
        _tc2_body,
        grid=(_GRID,),
        in_specs=[
            pl.BlockSpec((_NCORES, _RB, _DHID), lambda i: (0, i, 0)),
            pl.BlockSpec((_RB, _DHID), lambda i: (i, 0)),
            pl.BlockSpec((_NCORES, _RB, _DEGW), lambda i: (0, i, 0)),
            pl.BlockSpec((_DHID, _DHID), lambda i: (0, 0)),
            pl.BlockSpec((1, _DHID), lambda i: (0, 0)),
        ],
        out_specs=pl.BlockSpec((_RB, _DHID), lambda i: (i, 0)),
        out_shape=jax.ShapeDtypeStruct((_N, _DHID), jnp.float32),
    )(acc1, hh, deg_parts, W2p, b1r)


def _tc3_body(acc_ref, hh_ref, deg_ref, b2_ref, o_ref):
    dinv = _dinv_block(deg_ref)
    s = acc_ref[0] + acc_ref[1] + hh_ref[...]
    z = s * dinv + b2_ref[...]
    col = lax.broadcasted_iota(jnp.int32, (_RB, _DHID), 1)
    mask = col < _NCLS
    neg = jnp.full_like(z, -3.0e38)
    m = jnp.max(jnp.where(mask, z, neg), axis=1, keepdims=True)
    e = jnp.where(mask, jnp.exp(z - m), 0.0)
    lse = jnp.log(jnp.sum(e, axis=1, keepdims=True))
    o_ref[...] = (z - m - lse)[:, :_NCLS]


def _tc3(acc2, hh2, deg_parts, b2r):
    return pl.pallas_call(
        _tc3_body,
        grid=(_GRID,),
        in_specs=[
            pl.BlockSpec((_NCORES, _RB, _DHID), lambda i: (0, i, 0)),
            pl.BlockSpec((_RB, _DHID), lambda i: (i, 0)),
            pl.BlockSpec((_NCORES, _RB, _DEGW), lambda i: (0, i, 0)),
            pl.BlockSpec((1, _DHID), lambda i: (0, 0)),
        ],
        out_specs=pl.BlockSpec((_RB, _NCLS), lambda i: (i, 0)),
        out_shape=jax.ShapeDtypeStruct((_N, _NCLS), jnp.float32),
    )(acc2, hh2, deg_parts, b2r)


def kernel(x, edge_index, W1, b1, W2, b2):
    npad_e = _EPAD - _E
    pad_src = jnp.zeros((npad_e,), jnp.int32)
    pad_dst = _N + (jnp.arange(npad_e, dtype=jnp.int32) % (_NPAD - _N))
    src2d = jnp.concatenate(
        [edge_index[0].astype(jnp.int32), pad_src]).reshape(_NW * _NCH, _CH)
    dst2d = jnp.concatenate(
        [edge_index[1].astype(jnp.int32), pad_dst]).reshape(_NW * _NCH, _CH)
    W2p = jnp.pad(W2, ((0, 0), (0, _DHID - _NCLS)))
    b1r = b1.reshape(1, _DHID)
    b2r = jnp.pad(b2, (0, _DHID - _NCLS)).reshape(1, _DHID)
    ones = jnp.ones((_CH, _DEGW), jnp.float32)
    zeros8 = jnp.zeros((_RPT, _DEGW), jnp.float32)
    zeros16 = jnp.zeros((_RPT, _DHID), jnp.float32)

    deg_parts = _sc_degree(dst2d, ones, zeros8)
    hh = _tc1(x, W1, deg_parts)
    acc1 = _sc_scatter(hh, src2d, dst2d, zeros16)
    hh2 = _tc2(acc1, hh, deg_parts, W2p, b1r)
    acc2 = _sc_scatter(hh2, src2d, dst2d, zeros16)
    return _tc3(acc2, hh2, deg_parts, b2r)


# speedup vs baseline: 1.2926x; 1.1455x over previous
import functools

import jax
import jax.numpy as jnp
from jax import lax
from jax.experimental import pallas as pl
from jax.experimental.pallas import tpu as pltpu
from jax.experimental.pallas import tpu_sc as plsc

_N = 10000
_E = 320000
_DIN = 128
_DHID = 16
_NCLS = 10

_NPAD = 10240
_RB = 1024
_GRID = _NPAD // _RB

_NCORES = 2
_NSUB = 16
_NW = _NCORES * _NSUB
_CH = 72
_NCH = 140
_EPAD = _NW * _NCH * _CH
_RPT = _NPAD // _NSUB
_DEGW = _DHID
_PK = 128 // _DHID
_PROWS = _NPAD // _PK
_RBP = _RB // _PK


def _mesh():
    return plsc.VectorSubcoreMesh(
        core_axis_name="c", subcore_axis_name="s",
        num_cores=_NCORES, num_subcores=_NSUB)


def _sc_degree(dst2d, ones, zeros16):
    @functools.partial(
        pl.kernel, mesh=_mesh(),
        compiler_params=pltpu.CompilerParams(use_tc_tiling_on_sc=False),
        out_type=jax.ShapeDtypeStruct((_NCORES, _NPAD, _DEGW), jnp.float32),
        scratch_types=[
            pltpu.VMEM((_NCH, _CH), jnp.int32),
            pltpu.VMEM((_CH, _DEGW), jnp.float32),
            pltpu.VMEM_SHARED((_NPAD, _DEGW), jnp.float32),
            [pltpu.SemaphoreType.DMA] * 2,
        ])
    def deg_kernel(dst_hbm, ones_hbm, z_hbm, out_hbm, didx, ones_v, acc, sem):
        cid = lax.axis_index("c")
        sid = lax.axis_index("s")
        wid = sid * _NCORES + cid
        pltpu.sync_copy(z_hbm, acc.at[pl.ds(sid * _RPT, _RPT)])
        pltpu.sync_copy(ones_hbm, ones_v)
        pltpu.sync_copy(dst_hbm.at[pl.ds(wid * _NCH, _NCH)], didx)
        plsc.subcore_barrier()

        pltpu.async_copy(ones_v, acc.at[didx.at[0]], sem[0], add=True)

        def chunk(j, c):
            e = 2 * j
            pltpu.async_copy(ones_v, acc.at[didx.at[e + 1]], sem[1],
                             add=True)
            pltpu.make_async_copy(ones_v, acc.at[didx.at[e]], sem[0]).wait()

            @pl.when(e + 2 < _NCH)
            def _():
                pltpu.async_copy(ones_v, acc.at[didx.at[e + 2]], sem[0],
                                 add=True)

            pltpu.make_async_copy(ones_v, acc.at[didx.at[e + 1]],
                                  sem[1]).wait()
            return c

        lax.fori_loop(0, _NCH // 2, chunk, None)
        plsc.subcore_barrier()
        pltpu.sync_copy(acc.at[pl.ds(sid * _RPT, _RPT)],
                        out_hbm.at[cid, pl.ds(sid * _RPT, _RPT)])

    return deg_kernel(dst2d, ones, zeros16)


def _sc_scatter(hh, src2d, dst2d, zeros16):
    @functools.partial(
        pl.kernel, mesh=_mesh(),
        compiler_params=pltpu.CompilerParams(use_tc_tiling_on_sc=False),
        out_type=jax.ShapeDtypeStruct((_NCORES, _NPAD, _DHID), jnp.float32),
        scratch_types=[
            pltpu.VMEM((_NCH, _CH), jnp.int32),
            pltpu.VMEM((_NCH, _CH), jnp.int32),
            pltpu.VMEM((4, _CH, _DHID), jnp.float32),
            pltpu.VMEM_SHARED((_NPAD, _DHID), jnp.float32),
            [pltpu.SemaphoreType.DMA] * 4,
            [pltpu.SemaphoreType.DMA] * 4,
        ])
    def scat_kernel(hh_hbm, src_hbm, dst_hbm, z_hbm, out_hbm,
                    sidx, didx, rows, acc, gsem, ssem):
        cid = lax.axis_index("c")
        sid = lax.axis_index("s")
        wid = sid * _NCORES + cid
        pltpu.sync_copy(z_hbm, acc.at[pl.ds(sid * _RPT, _RPT)])
        pltpu.sync_copy(src_hbm.at[pl.ds(wid * _NCH, _NCH)], sidx)
        pltpu.sync_copy(dst_hbm.at[pl.ds(wid * _NCH, _NCH)], didx)
        plsc.subcore_barrier()

        def gat(c, k):
            return pltpu.make_async_copy(hh_hbm.at[sidx.at[c]], rows.at[k],
                                         gsem[k])

        def sca(c, k):
            return pltpu.make_async_copy(rows.at[k], acc.at[didx.at[c]],
                                         ssem[k])

        for k in range(4):
            pltpu.async_copy(hh_hbm.at[sidx.at[k]], rows.at[k], gsem[k])

        def quad(j, carry):
            c0 = 4 * j
            for k in range(4):
                gat(c0 + k, k).wait()
                pltpu.async_copy(rows.at[k], acc.at[didx.at[c0 + k]],
                                 ssem[k], add=True)
            for k in range(4):
                sca(c0 + k, k).wait()
                pltpu.async_copy(hh_hbm.at[sidx.at[c0 + 4 + k]], rows.at[k],
                                 gsem[k])
            return carry

        lax.fori_loop(0, _NCH // 4 - 1, quad, None)
        c0 = _NCH - 4
        for k in range(4):
            gat(c0 + k, k).wait()
            pltpu.async_copy(rows.at[k], acc.at[didx.at[c0 + k]],
                             ssem[k], add=True)
        for k in range(4):
            sca(c0 + k, k).wait()
        plsc.subcore_barrier()
        pltpu.sync_copy(acc.at[pl.ds(sid * _RPT, _RPT)],
                        out_hbm.at[cid, pl.ds(sid * _RPT, _RPT)])

    return scat_kernel(hh, src2d, dst2d, zeros16)


def _dinv_packed(deg_ref):
    dsum = deg_ref[0] + deg_ref[1] + 1.0
    return lax.rsqrt(jnp.maximum(dsum, 1.0))


def _tc1_body(xp_ref, w1_ref, deg_ref, o_ref):
    hp = jnp.dot(xp_ref[...], w1_ref[...],
                 preferred_element_type=jnp.float32)
    o_ref[...] = hp * _dinv_packed(deg_ref)


def _tc1(xp, W1blk, degp):
    return pl.pallas_call(
        _tc1_body,
        grid=(_GRID,),
        in_specs=[
            pl.BlockSpec((_RBP, _PK * _DIN), lambda i: (i, 0)),
            pl.BlockSpec((_PK * _DIN, 128), lambda i: (0, 0)),
            pl.BlockSpec((_NCORES, _RBP, 128), lambda i: (0, i, 0)),
        ],
        out_specs=pl.BlockSpec((_RBP, 128), lambda i: (i, 0)),
        out_shape=jax.ShapeDtypeStruct((_PROWS, 128), jnp.float32),
    )(xp, W1blk, degp)


def _tc2_body(acc_ref, hh_ref, deg_ref, w2_ref, b1_ref, o_ref):
    dinv = _dinv_packed(deg_ref)
    s = acc_ref[0] + acc_ref[1] + hh_ref[...]
    h1 = jnp.maximum(s * dinv + b1_ref[...], 0.0)
    h2 = jnp.dot(h1, w2_ref[...], preferred_element_type=jnp.float32)
    o_ref[...] = h2 * dinv


def _tc2(acc1p, hhp, degp, W2blk, b1p):
    return pl.pallas_call(
        _tc2_body,
        grid=(_GRID,),
        in_specs=[
            pl.BlockSpec((_NCORES, _RBP, 128), lambda i: (0, i, 0)),
            pl.BlockSpec((_RBP, 128), lambda i: (i, 0)),
            pl.BlockSpec((_NCORES, _RBP, 128), lambda i: (0, i, 0)),
            pl.BlockSpec((128, 128), lambda i: (0, 0)),
            pl.BlockSpec((1, 128), lambda i: (0, 0)),
        ],
        out_specs=pl.BlockSpec((_RBP, 128), lambda i: (i, 0)),
        out_shape=jax.ShapeDtypeStruct((_PROWS, 128), jnp.float32),
    )(acc1p, hhp, degp, W2blk, b1p)


def _tc3_body(acc_ref, hh_ref, deg_ref, b2_ref, o_ref):
    dinv = _dinv_packed(deg_ref)
    s = acc_ref[0] + acc_ref[1] + hh_ref[...]
    zp = s * dinv + b2_ref[...]
    col = lax.broadcasted_iota(jnp.int32, (_RBP, _DHID), 1)
    mask = col < _NCLS
    outs = []
    for a in range(_PK):
        z = zp[:, a * _DHID:(a + 1) * _DHID]
        neg = jnp.full_like(z, -3.0e38)
        m = jnp.max(jnp.where(mask, z, neg), axis=1, keepdims=True)
        e = jnp.where(mask, jnp.exp(z - m), 0.0)
        lse = jnp.log(jnp.sum(e, axis=1, keepdims=True))
        outs.append(z - m - lse)
    o_ref[...] = jnp.concatenate(outs, axis=1)


def _tc3(acc2p, hh2p, degp, b2p):
    return pl.pallas_call(
        _tc3_body,
        grid=(_GRID,),
        in_specs=[
            pl.BlockSpec((_NCORES, _RBP, 128), lambda i: (0, i, 0)),
            pl.BlockSpec((_RBP, 128), lambda i: (i, 0)),
            pl.BlockSpec((_NCORES, _RBP, 128), lambda i: (0, i, 0)),
            pl.BlockSpec((1, 128), lambda i: (0, 0)),
        ],
        out_specs=pl.BlockSpec((_RBP, 128), lambda i: (i, 0)),
        out_shape=jax.ShapeDtypeStruct((_PROWS, 128), jnp.float32),
    )(acc2p, hh2p, degp, b2p)


def kernel(x, edge_index, W1, b1, W2, b2):
    npad_e = _EPAD - _E
    pad_src = jnp.zeros((npad_e,), jnp.int32)
    pad_dst = _N + (jnp.arange(npad_e, dtype=jnp.int32) % (_NPAD - _N))
    src2d = jnp.concatenate(
        [edge_index[0].astype(jnp.int32), pad_src]).reshape(_NW * _NCH, _CH)
    dst2d = jnp.concatenate(
        [edge_index[1].astype(jnp.int32), pad_dst]).reshape(_NW * _NCH, _CH)
    W2p = jnp.pad(W2, ((0, 0), (0, _DHID - _NCLS)))
    W2blk = jnp.kron(jnp.eye(_PK, dtype=jnp.float32), W2p)
    W1blk = jnp.kron(jnp.eye(_PK, dtype=jnp.float32), W1)
    xp = jnp.pad(x, ((0, _NPAD - _N), (0, 0))).reshape(_PROWS, _PK * _DIN)
    b1p = jnp.tile(b1, _PK).reshape(1, 128)
    b2p = jnp.tile(jnp.pad(b2, (0, _DHID - _NCLS)), _PK).reshape(1, 128)
    ones = jnp.ones((_CH, _DEGW), jnp.float32)
    zeros16 = jnp.zeros((_RPT, _DHID), jnp.float32)

    deg_parts = _sc_degree(dst2d, ones, zeros16)
    degp = deg_parts.reshape(_NCORES, _PROWS, 128)
    hhp = _tc1(xp, W1blk, degp)
    acc1 = _sc_scatter(hhp.reshape(_NPAD, _DHID), src2d, dst2d, zeros16)
    hh2p = _tc2(acc1.reshape(_NCORES, _PROWS, 128), hhp, degp, W2blk, b1p)
    acc2 = _sc_scatter(hh2p.reshape(_NPAD, _DHID), src2d, dst2d, zeros16)
    outp = _tc3(acc2.reshape(_NCORES, _PROWS, 128), hh2p, degp, b2p)
    return outp.reshape(_NPAD, _DHID)[:_N, :_NCLS]


# speedup vs baseline: 1.6973x; 1.3131x over previous
import functools

import jax
import jax.numpy as jnp
from jax import lax
from jax.experimental import pallas as pl
from jax.experimental.pallas import tpu as pltpu
from jax.experimental.pallas import tpu_sc as plsc

_N = 10000
_E = 320000
_DIN = 128
_DHID = 16
_NCLS = 10

_NPAD = 10240
_RB = 1024
_GRID = _NPAD // _RB

_NCORES = 2
_NSUB = 16
_NW = _NCORES * _NSUB
_EPW = _E // _NW
_CH = 80
_NCH = _EPW // _CH
_RPT = _NPAD // _NSUB
_DEGW = _DHID
_PK = 128 // _DHID
_PROWS = _NPAD // _PK
_RBP = _RB // _PK


def _mesh():
    return plsc.VectorSubcoreMesh(
        core_axis_name="c", subcore_axis_name="s",
        num_cores=_NCORES, num_subcores=_NSUB)


def _sc_degree(ei, ones, zeros16):
    @functools.partial(
        pl.kernel, mesh=_mesh(),
        compiler_params=pltpu.CompilerParams(use_tc_tiling_on_sc=False),
        out_type=jax.ShapeDtypeStruct((_NCORES, _NPAD, _DEGW), jnp.float32),
        scratch_types=[
            pltpu.VMEM((_EPW,), jnp.int32),
            pltpu.VMEM((_CH, _DEGW), jnp.float32),
            pltpu.VMEM_SHARED((_NPAD, _DEGW), jnp.float32),
            [pltpu.SemaphoreType.DMA] * 2,
        ])
    def deg_kernel(ei_hbm, ones_hbm, z_hbm, out_hbm, didx, ones_v, acc, sem):
        cid = lax.axis_index("c")
        sid = lax.axis_index("s")
        wid = sid * _NCORES + cid
        pltpu.sync_copy(z_hbm, acc.at[pl.ds(sid * _RPT, _RPT)])
        pltpu.sync_copy(ones_hbm, ones_v)
        pltpu.sync_copy(ei_hbm.at[1, pl.ds(wid * _EPW, _EPW)], didx)
        plsc.subcore_barrier()

        def dst_at(c):
            return acc.at[didx.at[pl.ds(c * _CH, _CH)]]

        pltpu.async_copy(ones_v, dst_at(0), sem[0], add=True)

        def chunk(j, c):
            e = 2 * j
            pltpu.async_copy(ones_v, dst_at(e + 1), sem[1], add=True)
            pltpu.make_async_copy(ones_v, dst_at(e), sem[0]).wait()

            @pl.when(e + 2 < _NCH)
            def _():
                pltpu.async_copy(ones_v, dst_at(e + 2), sem[0], add=True)

            pltpu.make_async_copy(ones_v, dst_at(e + 1), sem[1]).wait()
            return c

        lax.fori_loop(0, _NCH // 2, chunk, None)
        last = _NCH - 1
        pltpu.make_async_copy(ones_v, dst_at(last), sem[0]).wait()
        plsc.subcore_barrier()
        pltpu.sync_copy(acc.at[pl.ds(sid * _RPT, _RPT)],
                        out_hbm.at[cid, pl.ds(sid * _RPT, _RPT)])

    return deg_kernel(ei, ones, zeros16)


def _sc_scatter(hh, ei, zeros16):
    @functools.partial(
        pl.kernel, mesh=_mesh(),
        compiler_params=pltpu.CompilerParams(use_tc_tiling_on_sc=False),
        out_type=jax.ShapeDtypeStruct((_NCORES, _NPAD, _DHID), jnp.float32),
        scratch_types=[
            pltpu.VMEM((_EPW,), jnp.int32),
            pltpu.VMEM((_EPW,), jnp.int32),
            pltpu.VMEM((4, _CH, _DHID), jnp.float32),
            pltpu.VMEM_SHARED((_NPAD, _DHID), jnp.float32),
            [pltpu.SemaphoreType.DMA] * 4,
            [pltpu.SemaphoreType.DMA] * 4,
        ])
    def scat_kernel(hh_hbm, ei_hbm, z_hbm, out_hbm,
                    sidx, didx, rows, acc, gsem, ssem):
        cid = lax.axis_index("c")
        sid = lax.axis_index("s")
        wid = sid * _NCORES + cid
        pltpu.sync_copy(z_hbm, acc.at[pl.ds(sid * _RPT, _RPT)])
        pltpu.sync_copy(ei_hbm.at[0, pl.ds(wid * _EPW, _EPW)], sidx)
        pltpu.sync_copy(ei_hbm.at[1, pl.ds(wid * _EPW, _EPW)], didx)
        plsc.subcore_barrier()

        def gat(c, k):
            return pltpu.make_async_copy(
                hh_hbm.at[sidx.at[pl.ds(c * _CH, _CH)]], rows.at[k], gsem[k])

        def sca(c, k):
            return pltpu.make_async_copy(
                rows.at[k], acc.at[didx.at[pl.ds(c * _CH, _CH)]], ssem[k])

        def start_gat(c, k):
            pltpu.async_copy(hh_hbm.at[sidx.at[pl.ds(c * _CH, _CH)]],
                             rows.at[k], gsem[k])

        def start_sca(c, k):
            pltpu.async_copy(rows.at[k],
                             acc.at[didx.at[pl.ds(c * _CH, _CH)]],
                             ssem[k], add=True)

        for k in range(4):
            start_gat(k, k)

        def quad(j, carry):
            c0 = 4 * j
            for k in range(4):
                gat(c0 + k, k).wait()
                start_sca(c0 + k, k)
            for k in range(4):
                sca(c0 + k, k).wait()
                start_gat(c0 + 4 + k, k)
            return carry

        lax.fori_loop(0, _NCH // 4 - 1, quad, None)
        c0 = 4 * (_NCH // 4 - 1)
        for k in range(4):
            gat(c0 + k, k).wait()
            start_sca(c0 + k, k)
        sca(c0, 0).wait()
        start_gat(_NCH - 1, 0)
        for k in range(1, 4):
            sca(c0 + k, k).wait()
        gat(_NCH - 1, 0).wait()
        start_sca(_NCH - 1, 0)
        sca(_NCH - 1, 0).wait()
        plsc.subcore_barrier()
        pltpu.sync_copy(acc.at[pl.ds(sid * _RPT, _RPT)],
                        out_hbm.at[cid, pl.ds(sid * _RPT, _RPT)])

    return scat_kernel(hh, ei, zeros16)


def _dinv_packed(deg_ref):
    dsum = deg_ref[0] + deg_ref[1] + 1.0
    return lax.rsqrt(jnp.maximum(dsum, 1.0))


def _tc1_body(xp_ref, w1_ref, deg_ref, o_ref):
    hp = jnp.dot(xp_ref[...], w1_ref[...],
                 preferred_element_type=jnp.float32)
    o_ref[...] = hp * _dinv_packed(deg_ref)


def _tc1(xp, W1blk, degp):
    return pl.pallas_call(
        _tc1_body,
        grid=(_GRID,),
        in_specs=[
            pl.BlockSpec((_RBP, _PK * _DIN), lambda i: (i, 0)),
            pl.BlockSpec((_PK * _DIN, 128), lambda i: (0, 0)),
            pl.BlockSpec((_NCORES, _RBP, 128), lambda i: (0, i, 0)),
        ],
        out_specs=pl.BlockSpec((_RBP, 128), lambda i: (i, 0)),
        out_shape=jax.ShapeDtypeStruct((_PROWS, 128), jnp.float32),
    )(xp, W1blk, degp)


def _tc2_body(acc_ref, hh_ref, deg_ref, w2_ref, b1_ref, o_ref):
    dinv = _dinv_packed(deg_ref)
    s = acc_ref[0] + acc_ref[1] + hh_ref[...]
    h1 = jnp.maximum(s * dinv + b1_ref[...], 0.0)
    h2 = jnp.dot(h1, w2_ref[...], preferred_element_type=jnp.float32)
    o_ref[...] = h2 * dinv


def _tc2(acc1p, hhp, degp, W2blk, b1p):
    return pl.pallas_call(
        _tc2_body,
        grid=(_GRID,),
        in_specs=[
            pl.BlockSpec((_NCORES, _RBP, 128), lambda i: (0, i, 0)),
            pl.BlockSpec((_RBP, 128), lambda i: (i, 0)),
            pl.BlockSpec((_NCORES, _RBP, 128), lambda i: (0, i, 0)),
            pl.BlockSpec((128, 128), lambda i: (0, 0)),
            pl.BlockSpec((1, 128), lambda i: (0, 0)),
        ],
        out_specs=pl.BlockSpec((_RBP, 128), lambda i: (i, 0)),
        out_shape=jax.ShapeDtypeStruct((_PROWS, 128), jnp.float32),
    )(acc1p, hhp, degp, W2blk, b1p)


def _tc3_body(acc_ref, hh_ref, deg_ref, b2_ref, o_ref):
    dinv = _dinv_packed(deg_ref)
    s = acc_ref[0] + acc_ref[1] + hh_ref[...]
    zp = s * dinv + b2_ref[...]
    col = lax.broadcasted_iota(jnp.int32, (_RBP, _DHID), 1)
    mask = col < _NCLS
    mparts = []
    for a in range(_PK):
        z = zp[:, a * _DHID:(a + 1) * _DHID]
        neg = jnp.full_like(z, -3.0e38)
        m = jnp.max(jnp.where(mask, z, neg), axis=1, keepdims=True)
        mparts.append(jnp.broadcast_to(m, (_RBP, _DHID)))
    mb = jnp.concatenate(mparts, axis=1)
    lane = lax.broadcasted_iota(jnp.int32, (_RBP, 128), 1)
    maskp = lax.rem(lane, _DHID) < _NCLS
    e = jnp.where(maskp, jnp.exp(zp - mb), 0.0)
    gi = lax.broadcasted_iota(jnp.int32, (128, 128), 0) // _DHID
    gj = lax.broadcasted_iota(jnp.int32, (128, 128), 1) // _DHID
    gmat = (gi == gj).astype(jnp.float32)
    gsum = jnp.dot(e, gmat, preferred_element_type=jnp.float32,
                   precision=lax.Precision.HIGHEST)
    o_ref[...] = zp - mb - jnp.log(gsum)


def _tc3(acc2p, hh2p, degp, b2p):
    return pl.pallas_call(
        _tc3_body,
        grid=(_GRID,),
        in_specs=[
            pl.BlockSpec((_NCORES, _RBP, 128), lambda i: (0, i, 0)),
            pl.BlockSpec((_RBP, 128), lambda i: (i, 0)),
            pl.BlockSpec((_NCORES, _RBP, 128), lambda i: (0, i, 0)),
            pl.BlockSpec((1, 128), lambda i: (0, 0)),
        ],
        out_specs=pl.BlockSpec((_RBP, 128), lambda i: (i, 0)),
        out_shape=jax.ShapeDtypeStruct((_PROWS, 128), jnp.float32),
    )(acc2p, hh2p, degp, b2p)


def kernel(x, edge_index, W1, b1, W2, b2):
    ei = edge_index.astype(jnp.int32)
    W2p = jnp.pad(W2, ((0, 0), (0, _DHID - _NCLS)))
    W2blk = jnp.kron(jnp.eye(_PK, dtype=jnp.float32), W2p)
    W1blk = jnp.kron(jnp.eye(_PK, dtype=jnp.float32), W1)
    xp = jnp.pad(x, ((0, _NPAD - _N), (0, 0))).reshape(_PROWS, _PK * _DIN)
    b1p = jnp.tile(b1, _PK).reshape(1, 128)
    b2p = jnp.tile(jnp.pad(b2, (0, _DHID - _NCLS)), _PK).reshape(1, 128)
    ones = jnp.ones((_CH, _DEGW), jnp.float32)
    zeros16 = jnp.zeros((_RPT, _DHID), jnp.float32)

    deg_parts = _sc_degree(ei, ones, zeros16)
    degp = deg_parts.reshape(_NCORES, _PROWS, 128)
    hhp = _tc1(xp, W1blk, degp)
    acc1 = _sc_scatter(hhp.reshape(_NPAD, _DHID), ei, zeros16)
    hh2p = _tc2(acc1.reshape(_NCORES, _PROWS, 128), hhp, degp, W2blk, b1p)
    acc2 = _sc_scatter(hh2p.reshape(_NPAD, _DHID), ei, zeros16)
    outp = _tc3(acc2.reshape(_NCORES, _PROWS, 128), hh2p, degp, b2p)
    return outp.reshape(_NPAD, _DHID)[:_N, :_NCLS]


# speedup vs baseline: 1.7184x; 1.0124x over previous
import functools

import jax
import jax.numpy as jnp
from jax import lax
from jax.experimental import pallas as pl
from jax.experimental.pallas import tpu as pltpu
from jax.experimental.pallas import tpu_sc as plsc

_N = 10000
_E = 320000
_DIN = 128
_DHID = 16
_NCLS = 10

_NPAD = 10240
_RB = 1024
_GRID = _NPAD // _RB

_NCORES = 2
_NSUB = 16
_NW = _NCORES * _NSUB
_EPW = _E // _NW
_CH = 80
_NCH = _EPW // _CH
_RPT = _NPAD // _NSUB
_DEGW = _DHID
_PK = 128 // _DHID
_PROWS = _NPAD // _PK
_RBP = _RB // _PK


def _mesh():
    return plsc.VectorSubcoreMesh(
        core_axis_name="c", subcore_axis_name="s",
        num_cores=_NCORES, num_subcores=_NSUB)


def _sc_degree(ei, ones, zeros16):
    @functools.partial(
        pl.kernel, mesh=_mesh(),
        compiler_params=pltpu.CompilerParams(use_tc_tiling_on_sc=False),
        out_type=jax.ShapeDtypeStruct((_NCORES, _NPAD, _DEGW), jnp.float32),
        scratch_types=[
            pltpu.VMEM((_EPW,), jnp.int32),
            pltpu.VMEM((_CH, _DEGW), jnp.float32),
            pltpu.VMEM_SHARED((_NPAD, _DEGW), jnp.float32),
            [pltpu.SemaphoreType.DMA] * 4,
        ])
    def deg_kernel(ei_hbm, ones_hbm, z_hbm, out_hbm, didx, ones_v, acc, sem):
        cid = lax.axis_index("c")
        sid = lax.axis_index("s")
        wid = sid * _NCORES + cid
        pltpu.sync_copy(z_hbm, acc.at[pl.ds(sid * _RPT, _RPT)])
        pltpu.sync_copy(ones_hbm, ones_v)
        pltpu.sync_copy(ei_hbm.at[1, pl.ds(wid * _EPW, _EPW)], didx)
        plsc.subcore_barrier()

        def dst_at(c):
            return acc.at[didx.at[pl.ds(c * _CH, _CH)]]

        for k in range(4):
            pltpu.async_copy(ones_v, dst_at(k), sem[k], add=True)

        def chunk(j, c):
            for k in range(4):
                pltpu.make_async_copy(ones_v, dst_at(4 * (j - 1) + k),
                                      sem[k]).wait()
                pltpu.async_copy(ones_v, dst_at(4 * j + k), sem[k],
                                 add=True)
            return c

        lax.fori_loop(1, _NCH // 4, chunk, None)
        c0 = 4 * (_NCH // 4 - 1)
        for k in range(4):
            pltpu.make_async_copy(ones_v, dst_at(c0 + k), sem[k]).wait()
        pltpu.async_copy(ones_v, dst_at(_NCH - 1), sem[0], add=True)
        pltpu.make_async_copy(ones_v, dst_at(_NCH - 1), sem[0]).wait()
        plsc.subcore_barrier()
        pltpu.sync_copy(acc.at[pl.ds(sid * _RPT, _RPT)],
                        out_hbm.at[cid, pl.ds(sid * _RPT, _RPT)])

    return deg_kernel(ei, ones, zeros16)


def _sc_scatter(hh, ei, zeros16):
    @functools.partial(
        pl.kernel, mesh=_mesh(),
        compiler_params=pltpu.CompilerParams(use_tc_tiling_on_sc=False),
        out_type=jax.ShapeDtypeStruct((_NCORES, _NPAD, _DHID), jnp.float32),
        scratch_types=[
            pltpu.VMEM((_EPW,), jnp.int32),
            pltpu.VMEM((_EPW,), jnp.int32),
            pltpu.VMEM((4, _CH, _DHID), jnp.float32),
            pltpu.VMEM_SHARED((_NPAD, _DHID), jnp.float32),
            [pltpu.SemaphoreType.DMA] * 4,
            [pltpu.SemaphoreType.DMA] * 4,
        ])
    def scat_kernel(hh_hbm, ei_hbm, z_hbm, out_hbm,
                    sidx, didx, rows, acc, gsem, ssem):
        cid = lax.axis_index("c")
        sid = lax.axis_index("s")
        wid = sid * _NCORES + cid
        pltpu.sync_copy(z_hbm, acc.at[pl.ds(sid * _RPT, _RPT)])
        pltpu.sync_copy(ei_hbm.at[0, pl.ds(wid * _EPW, _EPW)], sidx)
        pltpu.sync_copy(ei_hbm.at[1, pl.ds(wid * _EPW, _EPW)], didx)
        plsc.subcore_barrier()

        def gat(c, k):
            return pltpu.make_async_copy(
                hh_hbm.at[sidx.at[pl.ds(c * _CH, _CH)]], rows.at[k], gsem[k])

        def sca(c, k):
            return pltpu.make_async_copy(
                rows.at[k], acc.at[didx.at[pl.ds(c * _CH, _CH)]], ssem[k])

        def start_gat(c, k):
            pltpu.async_copy(hh_hbm.at[sidx.at[pl.ds(c * _CH, _CH)]],
                             rows.at[k], gsem[k])

        def start_sca(c, k):
            pltpu.async_copy(rows.at[k],
                             acc.at[didx.at[pl.ds(c * _CH, _CH)]],
                             ssem[k], add=True)

        for k in range(4):
            start_gat(k, k)

        def quad(j, carry):
            c0 = 4 * j
            for k in range(4):
                gat(c0 + k, k).wait()
                start_sca(c0 + k, k)
            for k in range(4):
                sca(c0 + k, k).wait()
                start_gat(c0 + 4 + k, k)
            return carry

        lax.fori_loop(0, _NCH // 4 - 1, quad, None)
        c0 = 4 * (_NCH // 4 - 1)
        for k in range(4):
            gat(c0 + k, k).wait()
            start_sca(c0 + k, k)
        sca(c0, 0).wait()
        start_gat(_NCH - 1, 0)
        for k in range(1, 4):
            sca(c0 + k, k).wait()
        gat(_NCH - 1, 0).wait()
        start_sca(_NCH - 1, 0)
        sca(_NCH - 1, 0).wait()
        plsc.subcore_barrier()
        pltpu.sync_copy(acc.at[pl.ds(sid * _RPT, _RPT)],
                        out_hbm.at[cid, pl.ds(sid * _RPT, _RPT)])

    return scat_kernel(hh, ei, zeros16)


def _dinv_packed(deg_ref):
    dsum = deg_ref[0] + deg_ref[1] + 1.0
    return lax.rsqrt(jnp.maximum(dsum, 1.0))


def _tc1_body(xp_ref, w1_ref, deg_ref, o_ref):
    hp = jnp.dot(xp_ref[...], w1_ref[...],
                 preferred_element_type=jnp.float32)
    o_ref[...] = hp * _dinv_packed(deg_ref)


def _tc1(xp, W1blk, degp):
    return pl.pallas_call(
        _tc1_body,
        grid=(_GRID,),
        in_specs=[
            pl.BlockSpec((_RBP, _PK * _DIN), lambda i: (i, 0)),
            pl.BlockSpec((_PK * _DIN, 128), lambda i: (0, 0)),
            pl.BlockSpec((_NCORES, _RBP, 128), lambda i: (0, i, 0)),
        ],
        out_specs=pl.BlockSpec((_RBP, 128), lambda i: (i, 0)),
        out_shape=jax.ShapeDtypeStruct((_PROWS, 128), jnp.float32),
    )(xp, W1blk, degp)


def _tc2_body(acc_ref, hh_ref, deg_ref, w2_ref, b1_ref, o_ref):
    dinv = _dinv_packed(deg_ref)
    s = acc_ref[0] + acc_ref[1] + hh_ref[...]
    h1 = jnp.maximum(s * dinv + b1_ref[...], 0.0)
    h2 = jnp.dot(h1, w2_ref[...], preferred_element_type=jnp.float32)
    o_ref[...] = h2 * dinv


def _tc2(acc1p, hhp, degp, W2blk, b1p):
    return pl.pallas_call(
        _tc2_body,
        grid=(_GRID,),
        in_specs=[
            pl.BlockSpec((_NCORES, _RBP, 128), lambda i: (0, i, 0)),
            pl.BlockSpec((_RBP, 128), lambda i: (i, 0)),
            pl.BlockSpec((_NCORES, _RBP, 128), lambda i: (0, i, 0)),
            pl.BlockSpec((128, 128), lambda i: (0, 0)),
            pl.BlockSpec((1, 128), lambda i: (0, 0)),
        ],
        out_specs=pl.BlockSpec((_RBP, 128), lambda i: (i, 0)),
        out_shape=jax.ShapeDtypeStruct((_PROWS, 128), jnp.float32),
    )(acc1p, hhp, degp, W2blk, b1p)


def _tc3_body(acc_ref, hh_ref, deg_ref, b2_ref, o_ref):
    dinv = _dinv_packed(deg_ref)
    s = acc_ref[0] + acc_ref[1] + hh_ref[...]
    zp = s * dinv + b2_ref[...]
    col = lax.broadcasted_iota(jnp.int32, (_RBP, _DHID), 1)
    mask = col < _NCLS
    mparts = []
    for a in range(_PK):
        z = zp[:, a * _DHID:(a + 1) * _DHID]
        neg = jnp.full_like(z, -3.0e38)
        m = jnp.max(jnp.where(mask, z, neg), axis=1, keepdims=True)
        mparts.append(jnp.broadcast_to(m, (_RBP, _DHID)))
    mb = jnp.concatenate(mparts, axis=1)
    lane = lax.broadcasted_iota(jnp.int32, (_RBP, 128), 1)
    maskp = lax.rem(lane, _DHID) < _NCLS
    e = jnp.where(maskp, jnp.exp(zp - mb), 0.0)
    gi = lax.broadcasted_iota(jnp.int32, (128, 128), 0) // _DHID
    gj = lax.broadcasted_iota(jnp.int32, (128, 128), 1) // _DHID
    gmat = (gi == gj).astype(jnp.float32)
    gsum = jnp.dot(e, gmat, preferred_element_type=jnp.float32,
                   precision=lax.Precision.HIGHEST)
    o_ref[...] = zp - mb - jnp.log(gsum)


def _tc3(acc2p, hh2p, degp, b2p):
    return pl.pallas_call(
        _tc3_body,
        grid=(_GRID,),
        in_specs=[
            pl.BlockSpec((_NCORES, _RBP, 128), lambda i: (0, i, 0)),
            pl.BlockSpec((_RBP, 128), lambda i: (i, 0)),
            pl.BlockSpec((_NCORES, _RBP, 128), lambda i: (0, i, 0)),
            pl.BlockSpec((1, 128), lambda i: (0, 0)),
        ],
        out_specs=pl.BlockSpec((_RBP, 128), lambda i: (i, 0)),
        out_shape=jax.ShapeDtypeStruct((_PROWS, 128), jnp.float32),
    )(acc2p, hh2p, degp, b2p)


def kernel(x, edge_index, W1, b1, W2, b2):
    ei = edge_index.astype(jnp.int32)
    W2p = jnp.pad(W2, ((0, 0), (0, _DHID - _NCLS)))
    W2blk = jnp.kron(jnp.eye(_PK, dtype=jnp.float32), W2p)
    W1blk = jnp.kron(jnp.eye(_PK, dtype=jnp.float32), W1)
    xp = jnp.pad(x.reshape(_N * _DIN), (0, (_NPAD - _N) * _DIN)).reshape(
        _PROWS, _PK * _DIN)
    b1p = jnp.tile(b1, _PK).reshape(1, 128)
    b2p = jnp.tile(jnp.pad(b2, (0, _DHID - _NCLS)), _PK).reshape(1, 128)
    ones = jnp.ones((_CH, _DEGW), jnp.float32)
    zeros16 = jnp.zeros((_RPT, _DHID), jnp.float32)

    deg_parts = _sc_degree(ei, ones, zeros16)
    degp = deg_parts.reshape(_NCORES, _PROWS, 128)
    hhp = _tc1(xp, W1blk, degp)
    acc1 = _sc_scatter(hhp.reshape(_NPAD, _DHID), ei, zeros16)
    hh2p = _tc2(acc1.reshape(_NCORES, _PROWS, 128), hhp, degp, W2blk, b1p)
    acc2 = _sc_scatter(hh2p.reshape(_NPAD, _DHID), ei, zeros16)
    outp = _tc3(acc2.reshape(_NCORES, _PROWS, 128), hh2p, degp, b2p)
    return outp.reshape(_NPAD, _DHID)[:_N, :_NCLS]


# speedup vs baseline: 1.9091x; 1.1110x over previous
import functools

import jax
import jax.numpy as jnp
from jax import lax
from jax.experimental import pallas as pl
from jax.experimental.pallas import tpu as pltpu
from jax.experimental.pallas import tpu_sc as plsc

_N = 10000
_E = 320000
_DIN = 128
_DHID = 16
_NCLS = 10

_NPAD = 10240
_RB = 1024
_GRID = _NPAD // _RB

_NCORES = 2
_NSUB = 16
_NW = _NCORES * _NSUB
_EPW = _E // _NW
_CH = 80
_NCH = _EPW // _CH
_SCH = 128
_SNCH = _EPW // _SCH
_STAIL = _EPW - _SNCH * _SCH
_RPT = _NPAD // _NSUB
_DEGW = _DHID
_PK = 128 // _DHID
_PROWS = _NPAD // _PK
_RBP = _RB // _PK


def _mesh():
    return plsc.VectorSubcoreMesh(
        core_axis_name="c", subcore_axis_name="s",
        num_cores=_NCORES, num_subcores=_NSUB)


def _sc_degree(ei, ones, zeros16):
    @functools.partial(
        pl.kernel, mesh=_mesh(),
        compiler_params=pltpu.CompilerParams(use_tc_tiling_on_sc=False),
        out_type=jax.ShapeDtypeStruct((_NCORES, _NPAD, _DEGW), jnp.float32),
        scratch_types=[
            pltpu.VMEM((_EPW,), jnp.int32),
            pltpu.VMEM((_CH, _DEGW), jnp.float32),
            pltpu.VMEM_SHARED((_NPAD, _DEGW), jnp.float32),
            [pltpu.SemaphoreType.DMA] * 4,
        ])
    def deg_kernel(ei_hbm, ones_hbm, z_hbm, out_hbm, didx, ones_v, acc, sem):
        cid = lax.axis_index("c")
        sid = lax.axis_index("s")
        wid = sid * _NCORES + cid
        pltpu.sync_copy(z_hbm, acc.at[pl.ds(sid * _RPT, _RPT)])
        pltpu.sync_copy(ones_hbm, ones_v)
        pltpu.sync_copy(ei_hbm.at[1, pl.ds(wid * _EPW, _EPW)], didx)
        plsc.subcore_barrier()

        def dst_at(c):
            return acc.at[didx.at[pl.ds(c * _CH, _CH)]]

        for k in range(4):
            pltpu.async_copy(ones_v, dst_at(k), sem[k], add=True)

        def chunk(j, c):
            for k in range(4):
                pltpu.make_async_copy(ones_v, dst_at(4 * (j - 1) + k),
                                      sem[k]).wait()
                pltpu.async_copy(ones_v, dst_at(4 * j + k), sem[k],
                                 add=True)
            return c

        lax.fori_loop(1, _NCH // 4, chunk, None)
        c0 = 4 * (_NCH // 4 - 1)
        for k in range(4):
            pltpu.make_async_copy(ones_v, dst_at(c0 + k), sem[k]).wait()
        pltpu.async_copy(ones_v, dst_at(_NCH - 1), sem[0], add=True)
        pltpu.make_async_copy(ones_v, dst_at(_NCH - 1), sem[0]).wait()
        plsc.subcore_barrier()
        pltpu.sync_copy(acc.at[pl.ds(sid * _RPT, _RPT)],
                        out_hbm.at[cid, pl.ds(sid * _RPT, _RPT)])

    return deg_kernel(ei, ones, zeros16)


def _sc_scatter(hh, ei, zeros16):
    @functools.partial(
        pl.kernel, mesh=_mesh(),
        compiler_params=pltpu.CompilerParams(use_tc_tiling_on_sc=False),
        out_type=jax.ShapeDtypeStruct((_NCORES, _NPAD, _DHID), jnp.float32),
        scratch_types=[
            pltpu.VMEM((_EPW,), jnp.int32),
            pltpu.VMEM((_EPW,), jnp.int32),
            pltpu.VMEM((4, _SCH, _DHID), jnp.float32),
            pltpu.VMEM((_STAIL, _DHID), jnp.float32),
            pltpu.VMEM_SHARED((_NPAD, _DHID), jnp.float32),
            [pltpu.SemaphoreType.DMA] * 4,
            [pltpu.SemaphoreType.DMA] * 4,
        ])
    def scat_kernel(hh_hbm, ei_hbm, z_hbm, out_hbm,
                    sidx, didx, rows, rowt, acc, gsem, ssem):
        cid = lax.axis_index("c")
        sid = lax.axis_index("s")
        wid = sid * _NCORES + cid
        pltpu.sync_copy(z_hbm, acc.at[pl.ds(sid * _RPT, _RPT)])
        pltpu.sync_copy(ei_hbm.at[0, pl.ds(wid * _EPW, _EPW)], sidx)
        pltpu.sync_copy(ei_hbm.at[1, pl.ds(wid * _EPW, _EPW)], didx)
        plsc.subcore_barrier()

        def gat(c, k):
            return pltpu.make_async_copy(
                hh_hbm.at[sidx.at[pl.ds(c * _SCH, _SCH)]], rows.at[k],
                gsem[k])

        def sca(c, k):
            return pltpu.make_async_copy(
                rows.at[k], acc.at[didx.at[pl.ds(c * _SCH, _SCH)]], ssem[k])

        def start_gat(c, k):
            pltpu.async_copy(hh_hbm.at[sidx.at[pl.ds(c * _SCH, _SCH)]],
                             rows.at[k], gsem[k])

        def start_sca(c, k):
            pltpu.async_copy(rows.at[k],
                             acc.at[didx.at[pl.ds(c * _SCH, _SCH)]],
                             ssem[k], add=True)

        for k in range(4):
            start_gat(k, k)

        def quad(j, carry):
            c0 = 4 * j
            for k in range(4):
                gat(c0 + k, k).wait()
                start_sca(c0 + k, k)
            for k in range(4):
                sca(c0 + k, k).wait()
                start_gat(c0 + 4 + k, k)
            return carry

        nq = _SNCH // 4 - 1
        lax.fori_loop(0, nq, quad, None)
        c0 = 4 * nq
        tb = _SNCH * _SCH
        for k in range(4):
            gat(c0 + k, k).wait()
            start_sca(c0 + k, k)
        for k in range(_SNCH - c0 - 4):
            sca(c0 + k, k).wait()
            start_gat(c0 + 4 + k, k)
        pltpu.async_copy(hh_hbm.at[sidx.at[pl.ds(tb, _STAIL)]], rowt,
                         gsem[2])
        for k in range(_SNCH - c0 - 4, 4):
            sca(c0 + k, k).wait()
        for k in range(_SNCH - c0 - 4):
            gat(c0 + 4 + k, k).wait()
            start_sca(c0 + 4 + k, k)
        pltpu.make_async_copy(hh_hbm.at[sidx.at[pl.ds(tb, _STAIL)]], rowt,
                              gsem[2]).wait()
        pltpu.sync_copy(rowt, acc.at[didx.at[pl.ds(tb, _STAIL)]], add=True)
        for k in range(_SNCH - c0 - 4):
            sca(c0 + 4 + k, k).wait()
        plsc.subcore_barrier()
        pltpu.sync_copy(acc.at[pl.ds(sid * _RPT, _RPT)],
                        out_hbm.at[cid, pl.ds(sid * _RPT, _RPT)])

    return scat_kernel(hh, ei, zeros16)


def _dinv_packed(deg_ref):
    dsum = deg_ref[0] + deg_ref[1] + 1.0
    return lax.rsqrt(jnp.maximum(dsum, 1.0))


def _tc1_body(xp_ref, w1_ref, deg_ref, o_ref):
    hp = jnp.dot(xp_ref[...], w1_ref[...],
                 preferred_element_type=jnp.float32)
    o_ref[...] = hp * _dinv_packed(deg_ref)


def _tc1(xp, W1blk, degp):
    return pl.pallas_call(
        _tc1_body,
        grid=(_GRID,),
        in_specs=[
            pl.BlockSpec((_RBP, _PK * _DIN), lambda i: (i, 0)),
            pl.BlockSpec((_PK * _DIN, 128), lambda i: (0, 0)),
            pl.BlockSpec((_NCORES, _RBP, 128), lambda i: (0, i, 0)),
        ],
        out_specs=pl.BlockSpec((_RBP, 128), lambda i: (i, 0)),
        out_shape=jax.ShapeDtypeStruct((_PROWS, 128), jnp.float32),
    )(xp, W1blk, degp)


def _tc2_body(acc_ref, hh_ref, deg_ref, w2_ref, b1_ref, o_ref):
    dinv = _dinv_packed(deg_ref)
    s = acc_ref[0] + acc_ref[1] + hh_ref[...]
    h1 = jnp.maximum(s * dinv + b1_ref[...], 0.0)
    h2 = jnp.dot(h1, w2_ref[...], preferred_element_type=jnp.float32)
    o_ref[...] = h2 * dinv


def _tc2(acc1p, hhp, degp, W2blk, b1p):
    return pl.pallas_call(
        _tc2_body,
        grid=(_GRID,),
        in_specs=[
            pl.BlockSpec((_NCORES, _RBP, 128), lambda i: (0, i, 0)),
            pl.BlockSpec((_RBP, 128), lambda i: (i, 0)),
            pl.BlockSpec((_NCORES, _RBP, 128), lambda i: (0, i, 0)),
            pl.BlockSpec((128, 128), lambda i: (0, 0)),
            pl.BlockSpec((1, 128), lambda i: (0, 0)),
        ],
        out_specs=pl.BlockSpec((_RBP, 128), lambda i: (i, 0)),
        out_shape=jax.ShapeDtypeStruct((_PROWS, 128), jnp.float32),
    )(acc1p, hhp, degp, W2blk, b1p)


def _tc3_body(acc_ref, hh_ref, deg_ref, b2_ref, o_ref):
    dinv = _dinv_packed(deg_ref)
    s = acc_ref[0] + acc_ref[1] + hh_ref[...]
    zp = s * dinv + b2_ref[...]
    col = lax.broadcasted_iota(jnp.int32, (_RBP, _DHID), 1)
    mask = col < _NCLS
    mparts = []
    for a in range(_PK):
        z = zp[:, a * _DHID:(a + 1) * _DHID]
        neg = jnp.full_like(z, -3.0e38)
        m = jnp.max(jnp.where(mask, z, neg), axis=1, keepdims=True)
        mparts.append(jnp.broadcast_to(m, (_RBP, _DHID)))
    mb = jnp.concatenate(mparts, axis=1)
    lane = lax.broadcasted_iota(jnp.int32, (_RBP, 128), 1)
    maskp = lax.rem(lane, _DHID) < _NCLS
    e = jnp.where(maskp, jnp.exp(zp - mb), 0.0)
    gi = lax.broadcasted_iota(jnp.int32, (128, 128), 0) // _DHID
    gj = lax.broadcasted_iota(jnp.int32, (128, 128), 1) // _DHID
    gmat = (gi == gj).astype(jnp.float32)
    gsum = jnp.dot(e, gmat, preferred_element_type=jnp.float32,
                   precision=lax.Precision.HIGHEST)
    o_ref[...] = zp - mb - jnp.log(gsum)


def _tc3(acc2p, hh2p, degp, b2p):
    return pl.pallas_call(
        _tc3_body,
        grid=(_GRID,),
        in_specs=[
            pl.BlockSpec((_NCORES, _RBP, 128), lambda i: (0, i, 0)),
            pl.BlockSpec((_RBP, 128), lambda i: (i, 0)),
            pl.BlockSpec((_NCORES, _RBP, 128), lambda i: (0, i, 0)),
            pl.BlockSpec((1, 128), lambda i: (0, 0)),
        ],
        out_specs=pl.BlockSpec((_RBP, 128), lambda i: (i, 0)),
        out_shape=jax.ShapeDtypeStruct((_PROWS, 128), jnp.float32),
    )(acc2p, hh2p, degp, b2p)


def kernel(x, edge_index, W1, b1, W2, b2):
    ei = edge_index.astype(jnp.int32)
    W2p = jnp.pad(W2, ((0, 0), (0, _DHID - _NCLS)))
    W2blk = jnp.kron(jnp.eye(_PK, dtype=jnp.float32), W2p)
    W1blk = jnp.kron(jnp.eye(_PK, dtype=jnp.float32), W1)
    xp = jnp.pad(x.reshape(_N * _DIN), (0, (_NPAD - _N) * _DIN)).reshape(
        _PROWS, _PK * _DIN)
    b1p = jnp.tile(b1, _PK).reshape(1, 128)
    b2p = jnp.tile(jnp.pad(b2, (0, _DHID - _NCLS)), _PK).reshape(1, 128)
    ones = jnp.ones((_CH, _DEGW), jnp.float32)
    zeros16 = jnp.zeros((_RPT, _DHID), jnp.float32)

    deg_parts = _sc_degree(ei, ones, zeros16)
    degp = deg_parts.reshape(_NCORES, _PROWS, 128)
    hhp = _tc1(xp, W1blk, degp)
    acc1 = _sc_scatter(hhp.reshape(_NPAD, _DHID), ei, zeros16)
    hh2p = _tc2(acc1.reshape(_NCORES, _PROWS, 128), hhp, degp, W2blk, b1p)
    acc2 = _sc_scatter(hh2p.reshape(_NPAD, _DHID), ei, zeros16)
    outp = _tc3(acc2.reshape(_NCORES, _PROWS, 128), hh2p, degp, b2p)
    return outp.reshape(_NPAD, _DHID)[:_N, :_NCLS]
